# baseline (speedup 1.0000x reference)
import functools

import jax
import jax.numpy as jnp
from jax import lax
from jax.experimental import pallas as pl
from jax.experimental.pallas import tpu as pltpu
from jax.experimental.pallas import tpu_sc as plsc

BATCH, HIST = 4096, 20
VOCAB, D_IN = 1000, 64

_NC, _NS = 2, 16
_NW = _NC * _NS
_B_TOT = BATCH * HIST
_ROWS_PER_W = _B_TOT // _NW
_CHUNK = 640
_NCHUNKS = _ROWS_PER_W // _CHUNK
_NBB = 512


def _sc_gather_body(emb_hbm, idx_hbm, h_hbm,
                    idx_v, buf0, buf1, gsem0, gsem1, wsem0, wsem1):
    wid = lax.axis_index("s") * _NC + lax.axis_index("c")
    base = wid * _ROWS_PER_W
    pltpu.sync_copy(idx_hbm.at[pl.ds(base, _ROWS_PER_W)], idx_v)

    def gather(c, buf, sem):
        return pltpu.make_async_copy(
            emb_hbm.at[idx_v.at[pl.ds(c * _CHUNK, _CHUNK)]], buf, sem)

    def write(c, buf, sem):
        return pltpu.make_async_copy(
            buf, h_hbm.at[pl.ds(base + c * _CHUNK, _CHUNK)], sem)

    gather(0, buf0, gsem0).start()

    def step(g, carry):
        c0 = 2 * g
        gather(0, buf0, gsem0).wait()

        @pl.when(g > 0)
        def _():
            write(0, buf1, wsem1).wait()

        gather(c0 + 1, buf1, gsem1).start()
        write(c0, buf0, wsem0).start()
        gather(0, buf1, gsem1).wait()
        write(0, buf0, wsem0).wait()

        @pl.when(g + 1 < _NCHUNKS // 2)
        def _():
            gather(c0 + 2, buf0, gsem0).start()

        write(c0 + 1, buf1, wsem1).start()
        return carry

    lax.fori_loop(0, _NCHUNKS // 2, step, 0)
    write(0, buf1, wsem1).wait()


_sc_gather = functools.partial(
    pl.kernel,
    out_type=jax.ShapeDtypeStruct((_B_TOT, D_IN), jnp.bfloat16),
    mesh=plsc.VectorSubcoreMesh(
        core_axis_name="c", subcore_axis_name="s",
        num_cores=_NC, num_subcores=_NS),
    scratch_types=[
        pltpu.VMEM((_ROWS_PER_W,), jnp.int32),
        pltpu.VMEM((_CHUNK, D_IN), jnp.bfloat16),
        pltpu.VMEM((_CHUNK, D_IN), jnp.bfloat16),
        pltpu.SemaphoreType.DMA,
        pltpu.SemaphoreType.DMA,
        pltpu.SemaphoreType.DMA,
        pltpu.SemaphoreType.DMA,
    ],
    compiler_params=pltpu.CompilerParams(use_tc_tiling_on_sc=False),
)(_sc_gather_body)


def _head_body(h_ref, w_ref, b_ref, out_ref):
    acc = lax.dot_general(
        w_ref[...], h_ref[...],
        dimension_numbers=(((1,), (1,)), ((), ())),
        preferred_element_type=jnp.float32,
    ) + b_ref[...]
    out_ref[...] = acc.reshape(1, VOCAB, _NBB)


def kernel(x, emb_table, W_head, b_head):
    h = _sc_gather(emb_table.astype(jnp.bfloat16), x.T.reshape(-1))
    out = pl.pallas_call(
        _head_body,
        grid=(HIST, BATCH // _NBB),
        in_specs=[
            pl.BlockSpec((_NBB, D_IN),
                         lambda h_i, nb: (h_i * (BATCH // _NBB) + nb, 0)),
            pl.BlockSpec((VOCAB, D_IN), lambda h_i, nb: (0, 0)),
            pl.BlockSpec((VOCAB, 1), lambda h_i, nb: (0, 0)),
        ],
        out_specs=pl.BlockSpec((1, VOCAB, _NBB), lambda h_i, nb: (h_i, 0, nb)),
        out_shape=jax.ShapeDtypeStruct((HIST, VOCAB, BATCH), jnp.float32),
    )(h, W_head.astype(jnp.bfloat16), b_head.reshape(VOCAB, 1))
    return jnp.transpose(out, (2, 0, 1))


# speedup vs baseline: 1.2395x; 1.2395x over previous
import functools

import jax
import jax.numpy as jnp
from jax import lax
from jax.experimental import pallas as pl
from jax.experimental.pallas import tpu as pltpu
from jax.experimental.pallas import tpu_sc as plsc

BATCH, HIST = 4096, 20
VOCAB, D_IN = 1000, 64

_NC, _NS = 2, 16
_NW = _NC * _NS
_B_TOT = BATCH * HIST
_ROWS_PER_W = _B_TOT // _NW
_CHUNK = 640
_NCHUNKS = _ROWS_PER_W // _CHUNK
_NBB = 1024


def _sc_gather_body(emb_hbm, idx_hbm, h_hbm,
                    idx_v, buf0, buf1, gsem0, gsem1, wsem0, wsem1):
    wid = lax.axis_index("s") * _NC + lax.axis_index("c")
    base = wid * _ROWS_PER_W
    pltpu.sync_copy(idx_hbm.at[pl.ds(base, _ROWS_PER_W)], idx_v)

    def gather(c, buf, sem):
        return pltpu.make_async_copy(
            emb_hbm.at[idx_v.at[pl.ds(c * _CHUNK, _CHUNK)]], buf, sem)

    def write(c, buf, sem):
        return pltpu.make_async_copy(
            buf, h_hbm.at[pl.ds(base + c * _CHUNK, _CHUNK)], sem)

    gather(0, buf0, gsem0).start()

    def step(g, carry):
        c0 = 2 * g
        gather(0, buf0, gsem0).wait()

        @pl.when(g > 0)
        def _():
            write(0, buf1, wsem1).wait()

        gather(c0 + 1, buf1, gsem1).start()
        write(c0, buf0, wsem0).start()
        gather(0, buf1, gsem1).wait()
        write(0, buf0, wsem0).wait()

        @pl.when(g + 1 < _NCHUNKS // 2)
        def _():
            gather(c0 + 2, buf0, gsem0).start()

        write(c0 + 1, buf1, wsem1).start()
        return carry

    lax.fori_loop(0, _NCHUNKS // 2, step, 0)
    write(0, buf1, wsem1).wait()


_sc_gather = functools.partial(
    pl.kernel,
    out_type=jax.ShapeDtypeStruct((_B_TOT, D_IN), jnp.float32),
    mesh=plsc.VectorSubcoreMesh(
        core_axis_name="c", subcore_axis_name="s",
        num_cores=_NC, num_subcores=_NS),
    scratch_types=[
        pltpu.VMEM((_ROWS_PER_W,), jnp.int32),
        pltpu.VMEM((_CHUNK, D_IN), jnp.float32),
        pltpu.VMEM((_CHUNK, D_IN), jnp.float32),
        pltpu.SemaphoreType.DMA,
        pltpu.SemaphoreType.DMA,
        pltpu.SemaphoreType.DMA,
        pltpu.SemaphoreType.DMA,
    ],
    compiler_params=pltpu.CompilerParams(use_tc_tiling_on_sc=False),
)(_sc_gather_body)


def _head_body(h_ref, w_ref, b_ref, out_ref):
    acc = lax.dot_general(
        w_ref[...], h_ref[...],
        dimension_numbers=(((1,), (1,)), ((), ())),
        preferred_element_type=jnp.float32,
    ) + b_ref[...]
    out_ref[...] = acc.reshape(1, VOCAB, _NBB)


def kernel(x, emb_table, W_head, b_head):
    h = _sc_gather(emb_table, x.T.reshape(-1))
    out = pl.pallas_call(
        _head_body,
        grid=(HIST, BATCH // _NBB),
        in_specs=[
            pl.BlockSpec((_NBB, D_IN),
                         lambda h_i, nb: (h_i * (BATCH // _NBB) + nb, 0)),
            pl.BlockSpec((VOCAB, D_IN), lambda h_i, nb: (0, 0)),
            pl.BlockSpec((VOCAB, 1), lambda h_i, nb: (0, 0)),
        ],
        out_specs=pl.BlockSpec((1, VOCAB, _NBB), lambda h_i, nb: (h_i, 0, nb)),
        out_shape=jax.ShapeDtypeStruct((HIST, VOCAB, BATCH), jnp.float32),
    )(h, W_head, b_head.reshape(VOCAB, 1))
    return jnp.transpose(out, (2, 0, 1))


# speedup vs baseline: 1.3618x; 1.0987x over previous
import functools

import jax
import jax.numpy as jnp
from jax import lax
from jax.experimental import pallas as pl
from jax.experimental.pallas import tpu as pltpu
from jax.experimental.pallas import tpu_sc as plsc

BATCH, HIST = 4096, 20
VOCAB, D_IN = 1000, 64

_NC, _NS = 2, 16
_NW = _NC * _NS
_B_TOT = BATCH * HIST
_ROWS_PER_W = _B_TOT // _NW
_CHUNK = 640
_NCHUNKS = _ROWS_PER_W // _CHUNK
_NBB = 2048


def _sc_gather_body(emb_hbm, idx_hbm, h_hbm,
                    idx_v, buf0, buf1, gsem0, gsem1, wsem0, wsem1):
    wid = lax.axis_index("s") * _NC + lax.axis_index("c")
    base = wid * _ROWS_PER_W
    pltpu.sync_copy(idx_hbm.at[pl.ds(base, _ROWS_PER_W)], idx_v)

    def gather(c, buf, sem):
        return pltpu.make_async_copy(
            emb_hbm.at[idx_v.at[pl.ds(c * _CHUNK, _CHUNK)]], buf, sem)

    def write(c, buf, sem):
        return pltpu.make_async_copy(
            buf, h_hbm.at[pl.ds(base + c * _CHUNK, _CHUNK)], sem)

    gather(0, buf0, gsem0).start()

    def step(g, carry):
        c0 = 2 * g
        gather(0, buf0, gsem0).wait()

        @pl.when(g > 0)
        def _():
            write(0, buf1, wsem1).wait()

        gather(c0 + 1, buf1, gsem1).start()
        write(c0, buf0, wsem0).start()
        gather(0, buf1, gsem1).wait()
        write(0, buf0, wsem0).wait()

        @pl.when(g + 1 < _NCHUNKS // 2)
        def _():
            gather(c0 + 2, buf0, gsem0).start()

        write(c0 + 1, buf1, wsem1).start()
        return carry

    lax.fori_loop(0, _NCHUNKS // 2, step, 0)
    write(0, buf1, wsem1).wait()


_sc_gather = functools.partial(
    pl.kernel,
    out_type=jax.ShapeDtypeStruct((_B_TOT, D_IN), jnp.float32),
    mesh=plsc.VectorSubcoreMesh(
        core_axis_name="c", subcore_axis_name="s",
        num_cores=_NC, num_subcores=_NS),
    scratch_types=[
        pltpu.VMEM((_ROWS_PER_W,), jnp.int32),
        pltpu.VMEM((_CHUNK, D_IN), jnp.float32),
        pltpu.VMEM((_CHUNK, D_IN), jnp.float32),
        pltpu.SemaphoreType.DMA,
        pltpu.SemaphoreType.DMA,
        pltpu.SemaphoreType.DMA,
        pltpu.SemaphoreType.DMA,
    ],
    compiler_params=pltpu.CompilerParams(use_tc_tiling_on_sc=False),
)(_sc_gather_body)


def _head_body(h_ref, w_ref, b_ref, out_ref):
    acc = lax.dot_general(
        w_ref[...], h_ref[...],
        dimension_numbers=(((1,), (1,)), ((), ())),
        preferred_element_type=jnp.float32,
    ) + b_ref[...]
    out_ref[...] = acc.reshape(1, VOCAB, _NBB)


def kernel(x, emb_table, W_head, b_head):
    h = _sc_gather(emb_table, x.T.reshape(-1))
    out = pl.pallas_call(
        _head_body,
        grid=(HIST, BATCH // _NBB),
        in_specs=[
            pl.BlockSpec((_NBB, D_IN),
                         lambda h_i, nb: (h_i * (BATCH // _NBB) + nb, 0)),
            pl.BlockSpec((VOCAB, D_IN), lambda h_i, nb: (0, 0)),
            pl.BlockSpec((VOCAB, 1), lambda h_i, nb: (0, 0)),
        ],
        out_specs=pl.BlockSpec((1, VOCAB, _NBB), lambda h_i, nb: (h_i, 0, nb)),
        out_shape=jax.ShapeDtypeStruct((HIST, VOCAB, BATCH), jnp.float32),
    )(h, W_head, b_head.reshape(VOCAB, 1))
    return jnp.transpose(out, (2, 0, 1))


# speedup vs baseline: 1.3732x; 1.0084x over previous
import functools

import jax
import jax.numpy as jnp
from jax import lax
from jax.experimental import pallas as pl
from jax.experimental.pallas import tpu as pltpu
from jax.experimental.pallas import tpu_sc as plsc

BATCH, HIST = 4096, 20
VOCAB, D_IN = 1000, 64

_NC, _NS = 2, 16
_NW = _NC * _NS
_B_TOT = BATCH * HIST
_ROWS_PER_W = _B_TOT // _NW
_CHUNK = 640
_NCHUNKS = _ROWS_PER_W // _CHUNK
_NBB = 4096


def _sc_gather_body(emb_hbm, idx_hbm, h_hbm,
                    idx_v, buf0, buf1, gsem0, gsem1, wsem0, wsem1):
    wid = lax.axis_index("s") * _NC + lax.axis_index("c")
    base = wid * _ROWS_PER_W
    pltpu.sync_copy(idx_hbm.at[pl.ds(base, _ROWS_PER_W)], idx_v)

    def gather(c, buf, sem):
        return pltpu.make_async_copy(
            emb_hbm.at[idx_v.at[pl.ds(c * _CHUNK, _CHUNK)]], buf, sem)

    def write(c, buf, sem):
        return pltpu.make_async_copy(
            buf, h_hbm.at[pl.ds(base + c * _CHUNK, _CHUNK)], sem)

    gather(0, buf0, gsem0).start()

    def step(g, carry):
        c0 = 2 * g
        gather(0, buf0, gsem0).wait()

        @pl.when(g > 0)
        def _():
            write(0, buf1, wsem1).wait()

        gather(c0 + 1, buf1, gsem1).start()
        write(c0, buf0, wsem0).start()
        gather(0, buf1, gsem1).wait()
        write(0, buf0, wsem0).wait()

        @pl.when(g + 1 < _NCHUNKS // 2)
        def _():
            gather(c0 + 2, buf0, gsem0).start()

        write(c0 + 1, buf1, wsem1).start()
        return carry

    lax.fori_loop(0, _NCHUNKS // 2, step, 0)
    write(0, buf1, wsem1).wait()


_sc_gather = functools.partial(
    pl.kernel,
    out_type=jax.ShapeDtypeStruct((_B_TOT, D_IN), jnp.float32),
    mesh=plsc.VectorSubcoreMesh(
        core_axis_name="c", subcore_axis_name="s",
        num_cores=_NC, num_subcores=_NS),
    scratch_types=[
        pltpu.VMEM((_ROWS_PER_W,), jnp.int32),
        pltpu.VMEM((_CHUNK, D_IN), jnp.float32),
        pltpu.VMEM((_CHUNK, D_IN), jnp.float32),
        pltpu.SemaphoreType.DMA,
        pltpu.SemaphoreType.DMA,
        pltpu.SemaphoreType.DMA,
        pltpu.SemaphoreType.DMA,
    ],
    compiler_params=pltpu.CompilerParams(use_tc_tiling_on_sc=False),
)(_sc_gather_body)


def _head_body(h_ref, w_ref, b_ref, out_ref):
    acc = lax.dot_general(
        w_ref[...], h_ref[...],
        dimension_numbers=(((1,), (1,)), ((), ())),
        preferred_element_type=jnp.float32,
    ) + b_ref[...]
    out_ref[...] = acc.reshape(1, VOCAB, _NBB)


def kernel(x, emb_table, W_head, b_head):
    h = _sc_gather(emb_table, x.T.reshape(-1))
    out = pl.pallas_call(
        _head_body,
        grid=(HIST, BATCH // _NBB),
        in_specs=[
            pl.BlockSpec((_NBB, D_IN),
                         lambda h_i, nb: (h_i * (BATCH // _NBB) + nb, 0)),
            pl.BlockSpec((VOCAB, D_IN), lambda h_i, nb: (0, 0)),
            pl.BlockSpec((VOCAB, 1), lambda h_i, nb: (0, 0)),
        ],
        out_specs=pl.BlockSpec((1, VOCAB, _NBB), lambda h_i, nb: (h_i, 0, nb)),
        out_shape=jax.ShapeDtypeStruct((HIST, VOCAB, BATCH), jnp.float32),
    )(h, W_head, b_head.reshape(VOCAB, 1))
    return jnp.transpose(out, (2, 0, 1))


# speedup vs baseline: 1.4881x; 1.0837x over previous
import functools

import jax
import jax.numpy as jnp
from jax import lax
from jax.experimental import pallas as pl
from jax.experimental.pallas import tpu as pltpu
from jax.experimental.pallas import tpu_sc as plsc

BATCH, HIST = 4096, 20
VOCAB, D_IN = 1000, 64

_NC, _NS = 2, 16
_NW = _NC * _NS
_B_TOT = BATCH * HIST
_ROWS_PER_W = _B_TOT // _NW
_DPAD = 128
_CHUNK = 320
_NCHUNKS = _ROWS_PER_W // _CHUNK
_NBB = 4096


def _sc_gather_body(emb_hbm, idx_hbm, h_hbm,
                    idx_v, buf0, buf1, gsem0, gsem1, wsem0, wsem1):
    wid = lax.axis_index("s") * _NC + lax.axis_index("c")
    base = wid * _ROWS_PER_W
    pltpu.sync_copy(idx_hbm.at[pl.ds(base, _ROWS_PER_W)], idx_v)

    def gather(c, buf, sem):
        return pltpu.make_async_copy(
            emb_hbm.at[idx_v.at[pl.ds(c * _CHUNK, _CHUNK)]], buf, sem)

    def write(c, buf, sem):
        return pltpu.make_async_copy(
            buf, h_hbm.at[pl.ds(base + c * _CHUNK, _CHUNK)], sem)

    gather(0, buf0, gsem0).start()

    def step(g, carry):
        c0 = 2 * g
        gather(0, buf0, gsem0).wait()

        @pl.when(g > 0)
        def _():
            write(0, buf1, wsem1).wait()

        gather(c0 + 1, buf1, gsem1).start()
        write(c0, buf0, wsem0).start()
        gather(0, buf1, gsem1).wait()
        write(0, buf0, wsem0).wait()

        @pl.when(g + 1 < _NCHUNKS // 2)
        def _():
            gather(c0 + 2, buf0, gsem0).start()

        write(c0 + 1, buf1, wsem1).start()
        return carry

    lax.fori_loop(0, _NCHUNKS // 2, step, 0)
    write(0, buf1, wsem1).wait()


_sc_gather = functools.partial(
    pl.kernel,
    out_type=jax.ShapeDtypeStruct((_B_TOT, _DPAD), jnp.float32),
    mesh=plsc.VectorSubcoreMesh(
        core_axis_name="c", subcore_axis_name="s",
        num_cores=_NC, num_subcores=_NS),
    scratch_types=[
        pltpu.VMEM((_ROWS_PER_W,), jnp.int32),
        pltpu.VMEM((_CHUNK, _DPAD), jnp.float32),
        pltpu.VMEM((_CHUNK, _DPAD), jnp.float32),
        pltpu.SemaphoreType.DMA,
        pltpu.SemaphoreType.DMA,
        pltpu.SemaphoreType.DMA,
        pltpu.SemaphoreType.DMA,
    ],
    compiler_params=pltpu.CompilerParams(use_tc_tiling_on_sc=False),
)(_sc_gather_body)


def _head_body(h_ref, w_ref, b_ref, out_ref):
    acc = lax.dot_general(
        w_ref[...], h_ref[...],
        dimension_numbers=(((1,), (1,)), ((), ())),
        preferred_element_type=jnp.float32,
    ) + b_ref[...]
    out_ref[...] = acc.reshape(1, VOCAB, _NBB)


def kernel(x, emb_table, W_head, b_head):
    emb_pad = jnp.pad(emb_table, ((0, 0), (0, _DPAD - D_IN)))
    h = _sc_gather(emb_pad, x.T.reshape(-1))
    out = pl.pallas_call(
        _head_body,
        grid=(HIST, BATCH // _NBB),
        in_specs=[
            pl.BlockSpec((_NBB, _DPAD),
                         lambda h_i, nb: (h_i * (BATCH // _NBB) + nb, 0)),
            pl.BlockSpec((VOCAB, _DPAD), lambda h_i, nb: (0, 0)),
            pl.BlockSpec((VOCAB, 1), lambda h_i, nb: (0, 0)),
        ],
        out_specs=pl.BlockSpec((1, VOCAB, _NBB), lambda h_i, nb: (h_i, 0, nb)),
        out_shape=jax.ShapeDtypeStruct((HIST, VOCAB, BATCH), jnp.float32),
    )(h, jnp.pad(W_head, ((0, 0), (0, _DPAD - D_IN))),
      b_head.reshape(VOCAB, 1))
    return jnp.transpose(out, (2, 0, 1))
